# initial kernel scaffold (unmeasured)
import functools

import jax
import jax.numpy as jnp
from jax import lax
from jax.experimental import pallas as pl
from jax.experimental.pallas import tpu as pltpu

N_DEV = 4
M = 4096
N = 8192
M_PER = M // N_DEV
WC = 2048
NB = N // WC
N_HOPS = 2 * (N_DEV - 1)


def _body(x_ref, w_ref, sx_ref, sw_ref, out_ref,
          comm_ref, stage_ref, send_sems, recv_sems, out_sem):
    my = lax.axis_index("i")
    left = lax.rem(my - 1 + N_DEV, N_DEV)
    right = lax.rem(my + 1, N_DEV)
    scale = sx_ref[0] * sw_ref[0]

    barrier_sem = pltpu.get_barrier_semaphore()
    for nbr in (left, right):
        pl.semaphore_signal(barrier_sem, inc=1, device_id=(nbr,),
                            device_id_type=pl.DeviceIdType.MESH)
    pl.semaphore_wait(barrier_sem, 2)

    def partial_chunk(c, col):
        xc = x_ref[pl.ds(c * M_PER, M_PER), :]
        wc = w_ref[:, col:col + WC]
        return jnp.dot(xc, wc, preferred_element_type=jnp.float32)

    def epilogue_store(acc_f32, c, col):
        y = acc_f32 * scale
        stage_ref[...] = y * jax.nn.sigmoid(y)
        cp = pltpu.make_async_copy(
            stage_ref, out_ref.at[pl.ds(c * M_PER, M_PER), col:col + WC],
            out_sem)
        cp.start()
        cp.wait()

    for b in range(NB):
        col = b * WC
        comm_ref[0] = partial_chunk(my, col).astype(jnp.bfloat16)

        for h in range(N_HOPS):
            s_slot = h % 2
            r_slot = (h + 1) % 2
            sem = b * N_HOPS + h
            rdma = pltpu.make_async_remote_copy(
                src_ref=comm_ref.at[s_slot],
                dst_ref=comm_ref.at[r_slot],
                send_sem=send_sems.at[sem],
                recv_sem=recv_sems.at[sem],
                device_id=(right,),
                device_id_type=pl.DeviceIdType.MESH,
            )
            rdma.start()
            rdma.wait()

            if h < N_DEV - 1:
                c = lax.rem(my - h - 1 + N_DEV, N_DEV)
                acc = comm_ref[r_slot].astype(jnp.float32) + partial_chunk(c, col)
                comm_ref[r_slot] = acc.astype(jnp.bfloat16)
                if h == N_DEV - 2:
                    epilogue_store(acc, c, col)
            else:
                t = h - (N_DEV - 1)
                c = lax.rem(my - t + N_DEV, N_DEV)
                epilogue_store(comm_ref[r_slot].astype(jnp.float32), c, col)


def kernel(x, w_mat, scale_x, scale_w):
    xb = x.astype(jnp.bfloat16)
    wb = w_mat.astype(jnp.bfloat16)

    return pl.pallas_call(
        _body,
        out_shape=jax.ShapeDtypeStruct((M, N), jnp.float32),
        in_specs=[
            pl.BlockSpec(memory_space=pltpu.VMEM),
            pl.BlockSpec(memory_space=pltpu.VMEM),
            pl.BlockSpec(memory_space=pltpu.SMEM),
            pl.BlockSpec(memory_space=pltpu.SMEM),
        ],
        out_specs=pl.BlockSpec(memory_space=pltpu.ANY),
        scratch_shapes=[
            pltpu.VMEM((2, M_PER, WC), jnp.bfloat16),
            pltpu.VMEM((M_PER, WC), jnp.float32),
            pltpu.SemaphoreType.DMA((NB * N_HOPS,)),
            pltpu.SemaphoreType.DMA((NB * N_HOPS,)),
            pltpu.SemaphoreType.DMA,
        ],
        compiler_params=pltpu.CompilerParams(collective_id=0),
    )(xb, wb, scale_x, scale_w)


# baseline (device time: 1401704 ns/iter reference)
import functools

import jax
import jax.numpy as jnp
from jax import lax
from jax.experimental import pallas as pl
from jax.experimental.pallas import tpu as pltpu

N_DEV = 4
M = 4096
N = 8192
M_PER = M // N_DEV
WC = 2048
NB = N // WC
N_HOPS = 2 * (N_DEV - 1)


def _body(x_ref, w_ref, sx_ref, sw_ref, out_ref,
          comm_ref, stage_ref, send_sems, recv_sems, out_sem):
    my = lax.axis_index("i")
    left = lax.rem(my - 1 + N_DEV, N_DEV)
    right = lax.rem(my + 1, N_DEV)
    scale = sx_ref[0] * sw_ref[0]

    barrier_sem = pltpu.get_barrier_semaphore()
    for nbr in (left, right):
        pl.semaphore_signal(barrier_sem, inc=1, device_id=(nbr,),
                            device_id_type=pl.DeviceIdType.MESH)
    pl.semaphore_wait(barrier_sem, 2)

    def partial_chunk(c, col):
        xc = x_ref[pl.ds(c * M_PER, M_PER), :]
        wc = w_ref[:, col:col + WC]
        return jnp.dot(xc, wc, preferred_element_type=jnp.float32)

    def epilogue_store(acc_f32, c, col):
        y = acc_f32 * scale
        stage_ref[...] = y * jax.nn.sigmoid(y)
        cp = pltpu.make_async_copy(
            stage_ref, out_ref.at[pl.ds(c * M_PER, M_PER), col:col + WC],
            out_sem)
        cp.start()
        cp.wait()

    for b in range(NB):
        col = b * WC
        comm_ref[0] = partial_chunk(my, col).astype(jnp.bfloat16)

        for h in range(N_HOPS):
            s_slot = h % 2
            r_slot = (h + 1) % 2
            sem = b * N_HOPS + h
            rdma = pltpu.make_async_remote_copy(
                src_ref=comm_ref.at[s_slot],
                dst_ref=comm_ref.at[r_slot],
                send_sem=send_sems.at[sem],
                recv_sem=recv_sems.at[sem],
                device_id=(right,),
                device_id_type=pl.DeviceIdType.MESH,
            )
            rdma.start()
            rdma.wait()

            if h < N_DEV - 1:
                c = lax.rem(my - h - 1 + N_DEV, N_DEV)
                acc = comm_ref[r_slot].astype(jnp.float32) + partial_chunk(c, col)
                comm_ref[r_slot] = acc.astype(jnp.bfloat16)
                if h == N_DEV - 2:
                    epilogue_store(acc, c, col)
            else:
                t = h - (N_DEV - 1)
                c = lax.rem(my - t + N_DEV, N_DEV)
                epilogue_store(comm_ref[r_slot].astype(jnp.float32), c, col)


def kernel(x, w_mat, scale_x, scale_w):
    xb = x.astype(jnp.bfloat16)
    wb = w_mat.astype(jnp.bfloat16)

    return pl.pallas_call(
        _body,
        out_shape=jax.ShapeDtypeStruct((M, N), jnp.float32),
        in_specs=[
            pl.BlockSpec(memory_space=pltpu.VMEM),
            pl.BlockSpec(memory_space=pltpu.VMEM),
            pl.BlockSpec(memory_space=pltpu.SMEM),
            pl.BlockSpec(memory_space=pltpu.SMEM),
        ],
        out_specs=pl.BlockSpec(memory_space=pl.ANY),
        scratch_shapes=[
            pltpu.VMEM((2, M_PER, WC), jnp.bfloat16),
            pltpu.VMEM((M_PER, WC), jnp.float32),
            pltpu.SemaphoreType.DMA((NB * N_HOPS,)),
            pltpu.SemaphoreType.DMA((NB * N_HOPS,)),
            pltpu.SemaphoreType.DMA,
        ],
        compiler_params=pltpu.CompilerParams(
            collective_id=0, vmem_limit_bytes=60 * 1024 * 1024),
    )(xb, wb, scale_x, scale_w)


# device time: 742039 ns/iter; 1.8890x vs baseline; 1.8890x over previous
import jax
import jax.numpy as jnp
from jax import lax
from jax.experimental import pallas as pl
from jax.experimental.pallas import tpu as pltpu

N_DEV = 4
M = 4096
N = 8192
M_PER = M // N_DEV
WC = 2048
NP = N // (2 * WC)
N_HOPS = 2 * (N_DEV - 1)


def _body(x_ref, w_ref, sx_ref, sw_ref, out_ref,
          comm_r, comm_l, wblk_ref, stage_ref,
          send_r, recv_r, send_l, recv_l, w_sem, out_sem):
    my = lax.axis_index("i")
    left = lax.rem(my - 1 + N_DEV, N_DEV)
    right = lax.rem(my + 1, N_DEV)
    scale = sx_ref[0] * sw_ref[0]

    barrier_sem = pltpu.get_barrier_semaphore()
    for nbr in (left, right):
        pl.semaphore_signal(barrier_sem, inc=1, device_id=(nbr,),
                            device_id_type=pl.DeviceIdType.MESH)
    pl.semaphore_wait(barrier_sem, 2)

    def partial(c, lo):
        xc = x_ref[pl.ds(c * M_PER, M_PER), :]
        return jnp.dot(xc, wblk_ref[:, lo:lo + WC],
                       preferred_element_type=jnp.float32)

    def epilogue_store(acc_f32, c, col):
        y = acc_f32 * scale
        stage_ref[...] = y * jax.nn.sigmoid(y)
        cp = pltpu.make_async_copy(
            stage_ref, out_ref.at[pl.ds(c * M_PER, M_PER), col:col + WC],
            out_sem)
        cp.start()
        cp.wait()

    for p in range(NP):
        col_r = p * 2 * WC
        col_l = col_r + WC

        cpw = pltpu.make_async_copy(
            w_ref.at[:, col_r:col_r + 2 * WC], wblk_ref, w_sem)
        cpw.start()
        cpw.wait()

        comm_r[0] = partial(my, 0).astype(jnp.bfloat16)
        comm_l[0] = partial(my, WC).astype(jnp.bfloat16)

        pending = []
        own = []
        for h in range(N_HOPS):
            s_slot = h % 2
            r_slot = (h + 1) % 2
            si = p * N_HOPS + h
            rd_r = pltpu.make_async_remote_copy(
                src_ref=comm_r.at[s_slot], dst_ref=comm_r.at[r_slot],
                send_sem=send_r.at[si], recv_sem=recv_r.at[si],
                device_id=(right,), device_id_type=pl.DeviceIdType.MESH)
            rd_l = pltpu.make_async_remote_copy(
                src_ref=comm_l.at[s_slot], dst_ref=comm_l.at[r_slot],
                send_sem=send_l.at[si], recv_sem=recv_l.at[si],
                device_id=(left,), device_id_type=pl.DeviceIdType.MESH)
            rd_r.start()
            rd_l.start()

            for ref, slot, c, col in pending:
                epilogue_store(ref[slot].astype(jnp.float32), c, col)
            pending = []
            for acc, c, col in own:
                epilogue_store(acc, c, col)
            own = []
            if h < N_DEV - 1:
                c_r = lax.rem(my - h - 1 + N_DEV, N_DEV)
                c_l = lax.rem(my + h + 1, N_DEV)
                p_r = partial(c_r, 0)
                p_l = partial(c_l, WC)

            rd_r.wait()
            rd_l.wait()

            if h < N_DEV - 1:
                acc_r = comm_r[r_slot].astype(jnp.float32) + p_r
                comm_r[r_slot] = acc_r.astype(jnp.bfloat16)
                acc_l = comm_l[r_slot].astype(jnp.float32) + p_l
                comm_l[r_slot] = acc_l.astype(jnp.bfloat16)
                if h == N_DEV - 2:
                    own = [(acc_r, c_r, col_r), (acc_l, c_l, col_l)]
            else:
                t = h - (N_DEV - 1)
                c_r = lax.rem(my - t + N_DEV, N_DEV)
                c_l = lax.rem(my + t, N_DEV)
                pending = [(comm_r, r_slot, c_r, col_r),
                           (comm_l, r_slot, c_l, col_l)]

        for ref, slot, c, col in pending:
            epilogue_store(ref[slot].astype(jnp.float32), c, col)


def kernel(x, w_mat, scale_x, scale_w):
    xb = x.astype(jnp.bfloat16)
    wb = w_mat.astype(jnp.bfloat16)

    return pl.pallas_call(
        _body,
        out_shape=jax.ShapeDtypeStruct((M, N), jnp.float32),
        in_specs=[
            pl.BlockSpec(memory_space=pltpu.VMEM),
            pl.BlockSpec(memory_space=pl.ANY),
            pl.BlockSpec(memory_space=pltpu.SMEM),
            pl.BlockSpec(memory_space=pltpu.SMEM),
        ],
        out_specs=pl.BlockSpec(memory_space=pl.ANY),
        scratch_shapes=[
            pltpu.VMEM((2, M_PER, WC), jnp.bfloat16),
            pltpu.VMEM((2, M_PER, WC), jnp.bfloat16),
            pltpu.VMEM((1024, 2 * WC), jnp.bfloat16),
            pltpu.VMEM((M_PER, WC), jnp.float32),
            pltpu.SemaphoreType.DMA((NP * N_HOPS,)),
            pltpu.SemaphoreType.DMA((NP * N_HOPS,)),
            pltpu.SemaphoreType.DMA((NP * N_HOPS,)),
            pltpu.SemaphoreType.DMA((NP * N_HOPS,)),
            pltpu.SemaphoreType.DMA,
            pltpu.SemaphoreType.DMA,
        ],
        compiler_params=pltpu.CompilerParams(
            collective_id=0, vmem_limit_bytes=60 * 1024 * 1024),
    )(xb, wb, scale_x, scale_w)


# device time: 704858 ns/iter; 1.9886x vs baseline; 1.0527x over previous
import jax
import jax.numpy as jnp
from jax import lax
from jax.experimental import pallas as pl
from jax.experimental.pallas import tpu as pltpu

N_DEV = 4
M = 4096
N = 8192
H = M // N_DEV
WC = 2048
NP = N // (2 * WC)
N_HOPS = 2 * (N_DEV - 1)


def _body(x_ref, w_ref, sx_ref, sw_ref, out_ref,
          comm_r, comm_l, wblk_ref, pre_ref, stage_ref,
          send_r, recv_r, send_l, recv_l, w_sem, out_sem):
    my = lax.axis_index("i")
    left = lax.rem(my - 1 + N_DEV, N_DEV)
    right = lax.rem(my + 1, N_DEV)
    scale = sx_ref[0] * sw_ref[0]

    barrier_sem = pltpu.get_barrier_semaphore()
    for nbr in (left, right):
        pl.semaphore_signal(barrier_sem, inc=1, device_id=(nbr,),
                            device_id_type=pl.DeviceIdType.MESH)
    pl.semaphore_wait(barrier_sem, 2)

    def wblk_fetch(p):
        cp = pltpu.make_async_copy(
            w_ref.at[:, p * 2 * WC:(p + 1) * 2 * WC], wblk_ref, w_sem)
        cp.start()
        return cp

    def phalf(c, wcol):
        xc = x_ref[pl.ds(c * H, H), :]
        return jnp.dot(xc, wblk_ref[:, wcol:wcol + H],
                       preferred_element_type=jnp.float32)

    def epi(ring_ref, slot, half, c, col):
        y = ring_ref[slot, half].astype(jnp.float32) * scale
        stage_ref[...] = y * jax.nn.sigmoid(y)
        cp = pltpu.make_async_copy(
            stage_ref, out_ref.at[pl.ds(c * H, H), col:col + H], out_sem)
        cp.start()
        cp.wait()

    def cid(h):
        if h < N_DEV - 1:
            return (lax.rem(my - h - 1 + N_DEV, N_DEV),
                    lax.rem(my + h + 1, N_DEV))
        t = h - (N_DEV - 1)
        return lax.rem(my - t + N_DEV, N_DEV), lax.rem(my + t, N_DEV)

    def predots(h):
        c_r, c_l = cid(h)
        pre_ref[0] = phalf(c_r, 0).astype(jnp.bfloat16)
        pre_ref[1] = phalf(c_l, WC).astype(jnp.bfloat16)
        pre_ref[2] = phalf(c_r, H).astype(jnp.bfloat16)
        pre_ref[3] = phalf(c_l, WC + H).astype(jnp.bfloat16)

    inflight = {}

    def start_pair(p, h, half):
        s_slot = h % 2
        r_slot = (h + 1) % 2
        si = p * N_HOPS + h
        d_r = pltpu.make_async_remote_copy(
            src_ref=comm_r.at[s_slot, half], dst_ref=comm_r.at[r_slot, half],
            send_sem=send_r.at[half, si], recv_sem=recv_r.at[half, si],
            device_id=(right,), device_id_type=pl.DeviceIdType.MESH)
        d_l = pltpu.make_async_remote_copy(
            src_ref=comm_l.at[s_slot, half], dst_ref=comm_l.at[r_slot, half],
            send_sem=send_l.at[half, si], recv_sem=recv_l.at[half, si],
            device_id=(left,), device_id_type=pl.DeviceIdType.MESH)
        d_r.start()
        d_l.start()
        inflight[(h, half)] = (d_r, d_l)

    def seed(p, half):
        comm_r[0, half] = phalf(my, half * H).astype(jnp.bfloat16)
        comm_l[0, half] = phalf(my, WC + half * H).astype(jnp.bfloat16)
        start_pair(p, 0, half)

    wblk_fetch(0).wait()
    seed(0, 0)
    seed(0, 1)
    predots(0)
    tasks = []

    for p in range(NP):
        col_r = p * 2 * WC
        col_l = col_r + WC
        wblk_cp = None
        for h in range(N_HOPS):
            rs = h < N_DEV - 1
            r_slot = (h + 1) % 2
            c_r, c_l = cid(h)

            d_r, d_l = inflight.pop((h, 0))
            d_r.wait()
            d_l.wait()
            if rs:
                comm_r[r_slot, 0] = comm_r[r_slot, 0] + pre_ref[0]
                comm_l[r_slot, 0] = comm_l[r_slot, 0] + pre_ref[1]
            if h < N_HOPS - 1:
                start_pair(p, h + 1, 0)
            for t_ in tasks[:2]:
                epi(*t_)

            d_r, d_l = inflight.pop((h, 1))
            d_r.wait()
            d_l.wait()
            if rs:
                comm_r[r_slot, 1] = comm_r[r_slot, 1] + pre_ref[2]
                comm_l[r_slot, 1] = comm_l[r_slot, 1] + pre_ref[3]
            if h < N_HOPS - 1:
                start_pair(p, h + 1, 1)
            for t_ in tasks[2:]:
                epi(*t_)
            tasks = []
            if h + 1 < N_DEV - 1:
                predots(h + 1)
            if h == N_DEV - 2 and p < NP - 1:
                wblk_cp = wblk_fetch(p + 1)

            if h >= N_DEV - 2:
                tasks = [(comm_r, r_slot, 0, c_r, col_r),
                         (comm_l, r_slot, 0, c_l, col_l),
                         (comm_r, r_slot, 1, c_r, col_r + H),
                         (comm_l, r_slot, 1, c_l, col_l + H)]

        if p < NP - 1:
            wblk_cp.wait()
            epi(*tasks[0])
            epi(*tasks[1])
            seed(p + 1, 0)
            epi(*tasks[2])
            epi(*tasks[3])
            seed(p + 1, 1)
            predots(0)
            tasks = []
        else:
            for t_ in tasks:
                epi(*t_)


def kernel(x, w_mat, scale_x, scale_w):
    xb = x.astype(jnp.bfloat16)
    wb = w_mat.astype(jnp.bfloat16)

    return pl.pallas_call(
        _body,
        out_shape=jax.ShapeDtypeStruct((M, N), jnp.float32),
        in_specs=[
            pl.BlockSpec(memory_space=pltpu.VMEM),
            pl.BlockSpec(memory_space=pl.ANY),
            pl.BlockSpec(memory_space=pltpu.SMEM),
            pl.BlockSpec(memory_space=pltpu.SMEM),
        ],
        out_specs=pl.BlockSpec(memory_space=pl.ANY),
        scratch_shapes=[
            pltpu.VMEM((2, 2, H, H), jnp.bfloat16),
            pltpu.VMEM((2, 2, H, H), jnp.bfloat16),
            pltpu.VMEM((1024, 2 * WC), jnp.bfloat16),
            pltpu.VMEM((4, H, H), jnp.bfloat16),
            pltpu.VMEM((H, H), jnp.float32),
            pltpu.SemaphoreType.DMA((2, NP * N_HOPS)),
            pltpu.SemaphoreType.DMA((2, NP * N_HOPS)),
            pltpu.SemaphoreType.DMA((2, NP * N_HOPS)),
            pltpu.SemaphoreType.DMA((2, NP * N_HOPS)),
            pltpu.SemaphoreType.DMA,
            pltpu.SemaphoreType.DMA,
        ],
        compiler_params=pltpu.CompilerParams(
            collective_id=0, vmem_limit_bytes=52 * 1024 * 1024),
    )(xb, wb, scale_x, scale_w)
